# baseline (speedup 1.0000x reference)
import functools

import jax
import jax.numpy as jnp
from jax import lax
from jax.experimental import pallas as pl
from jax.experimental.pallas import tpu as pltpu
from jax.experimental.pallas import tpu_sc as plsc

NN = 10000
EE = 320000
D = 128
LD = 64

NC = 2
NS = 16
NW = NC * NS

EB = 128
NB = 80
EPT = NB * EB
EPAD = EPT * NW
EPC = EPAD // NC

NPAD = 10240
RPT = NPAD // NS
DEGW = 16

BR = 1024


def _sc_mesh():
    return plsc.VectorSubcoreMesh(
        core_axis_name="c", subcore_axis_name="s", num_cores=NC, num_subcores=NS)


def _deg_body(dst_h, out_h, acc, idx_d, ones_b, zero_b):
    c = lax.axis_index("c")
    s = lax.axis_index("s")
    one_v = jnp.ones((16,), jnp.float32)
    zero_v = jnp.zeros((16,), jnp.float32)
    for r in range(EB):
        ones_b[r, :] = one_v
        zero_b[r, :] = zero_v
    r0 = s * RPT
    def zloop(i, carry):
        pltpu.sync_copy(zero_b, acc.at[pl.ds(r0 + i * EB, EB)])
        return carry
    lax.fori_loop(0, RPT // EB, zloop, 0)
    plsc.subcore_barrier()
    base = c * EPC + s * EPT
    def eloop(b, carry):
        pltpu.sync_copy(dst_h.at[pl.ds(base + b * EB, EB)], idx_d)
        pltpu.sync_copy(ones_b, acc.at[idx_d], add=True)
        return carry
    lax.fori_loop(0, NB, eloop, 0)
    plsc.subcore_barrier()
    def oloop(k, carry):
        rr = r0 + k * EB
        pltpu.sync_copy(acc.at[pl.ds(rr, EB)], zero_b)
        pltpu.sync_copy(zero_b, out_h.at[pl.ds(c * NPAD + rr, EB)])
        return carry
    lax.fori_loop(0, RPT // EB, oloop, 0)


@jax.jit
def _deg_call(dst_p):
    return pl.kernel(
        _deg_body,
        out_type=jax.ShapeDtypeStruct((NC * NPAD, DEGW), jnp.float32),
        mesh=_sc_mesh(),
        scratch_types=[
            pltpu.VMEM_SHARED((NPAD, DEGW), jnp.float32),
            pltpu.VMEM((EB,), jnp.int32),
            pltpu.VMEM((EB, DEGW), jnp.float32),
            pltpu.VMEM((EB, DEGW), jnp.float32),
        ],
    )(dst_p)


EROWS = EPAD // EB


def _spmm_body(src_h, dst_h, u_h, out_h, acc, src_b, d_a, d_b, rows_a, rows_b,
               sem_a, sem_b, dsem_a, dsem_b):
    c = lax.axis_index("c")
    s = lax.axis_index("s")
    zero_v = jnp.zeros((16,), jnp.float32)
    for r in range(16):
        for j in range(8):
            rows_a[r, pl.ds(j * 16, 16)] = zero_v
    r0 = s * RPT
    zsrc = rows_a.at[pl.ds(0, 16)]
    def zloop(i, carry):
        pltpu.sync_copy(zsrc, acc.at[pl.ds(r0 + i * 16, 16)])
        return carry
    lax.fori_loop(0, RPT // 16, zloop, 0)
    base = c * EPC + s * EPT
    pltpu.sync_copy(src_h.at[pl.ds(base, EPT)], src_b)
    plsc.subcore_barrier()

    def gstart(b, rbuf, sem):
        pltpu.async_copy(u_h.at[src_b.at[pl.ds(b * EB, EB)]], rbuf, sem)

    def gwait(rbuf, sem):
        pltpu.make_async_copy(u_h.at[src_b.at[pl.ds(0, EB)]], rbuf, sem).wait()

    def dstart(b, dbuf, dsem):
        pltpu.async_copy(dst_h.at[pl.ds(base + b * EB, EB)], dbuf, dsem)

    def dwait(dbuf, dsem):
        pltpu.make_async_copy(dst_h.at[pl.ds(base, EB)], dbuf, dsem).wait()

    def scat(rbuf, dbuf):
        pltpu.sync_copy(rbuf, acc.at[dbuf], add=True)

    gstart(0, rows_a, sem_a)
    dstart(0, d_a, dsem_a)
    def eloop(g, carry):
        b0 = 2 * g
        gstart(b0 + 1, rows_b, sem_b)
        dstart(b0 + 1, d_b, dsem_b)
        gwait(rows_a, sem_a)
        dwait(d_a, dsem_a)
        scat(rows_a, d_a)
        gstart(b0 + 2, rows_a, sem_a)
        dstart(b0 + 2, d_a, dsem_a)
        gwait(rows_b, sem_b)
        dwait(d_b, dsem_b)
        scat(rows_b, d_b)
        return carry
    lax.fori_loop(0, NB // 2 - 1, eloop, 0)
    gstart(NB - 1, rows_b, sem_b)
    dstart(NB - 1, d_b, dsem_b)
    gwait(rows_a, sem_a)
    dwait(d_a, dsem_a)
    scat(rows_a, d_a)
    gwait(rows_b, sem_b)
    dwait(d_b, dsem_b)
    scat(rows_b, d_b)
    plsc.subcore_barrier()
    def oloop(k, carry):
        rr = r0 + k * EB
        pltpu.sync_copy(acc.at[pl.ds(rr, EB)], rows_a)
        pltpu.sync_copy(rows_a, out_h.at[pl.ds(c * NPAD + rr, EB)])
        return carry
    lax.fori_loop(0, RPT // EB, oloop, 0)


@jax.jit
def _spmm_call(src_p, dst_p, u):
    out = pl.kernel(
        _spmm_body,
        out_type=jax.ShapeDtypeStruct((NC * NPAD, D), jnp.float32),
        mesh=_sc_mesh(),
        scratch_types=[
            pltpu.VMEM_SHARED((NPAD, D), jnp.float32),
            pltpu.VMEM((NB * EB,), jnp.int32),
            pltpu.VMEM((EB,), jnp.int32),
            pltpu.VMEM((EB,), jnp.int32),
            pltpu.VMEM((EB, D), jnp.float32),
            pltpu.VMEM((EB, D), jnp.float32),
            pltpu.SemaphoreType.DMA,
            pltpu.SemaphoreType.DMA,
            pltpu.SemaphoreType.DMA,
            pltpu.SemaphoreType.DMA,
        ],
    )(src_p, dst_p, u)
    return out.reshape(NC, NPAD, D)


def _dinv_of(deg_ref):
    deg = deg_ref[0] + deg_ref[1]
    return lax.rsqrt(deg[:, :1] + 1.0)


def _tc1_body(deg_ref, x_ref, w_ref, u_ref):
    dinv = _dinv_of(deg_ref)
    y = jnp.dot(x_ref[...], w_ref[...], preferred_element_type=jnp.float32)
    u_ref[...] = dinv * y


def _tc2_body(deg_ref, p_ref, u1_ref, w_ref, b_ref, u2_ref):
    dinv = _dinv_of(deg_ref)
    agg = p_ref[0] + p_ref[1] + u1_ref[...]
    h = dinv * agg + b_ref[...]
    y = jnp.dot(h, w_ref[...], preferred_element_type=jnp.float32)
    u2_ref[...] = dinv * y


def _tc3_body(deg_ref, p_ref, u2_ref, b_ref, n_ref, w_ref,
              mean_ref, logvar_ref, z_ref, u3_ref):
    dinv = _dinv_of(deg_ref)
    ml = dinv * (p_ref[0] + p_ref[1] + u2_ref[...]) + b_ref[...]
    mean = ml[:, :LD]
    logvar = ml[:, LD:]
    z = n_ref[...] * jnp.exp(0.5 * logvar) + mean
    mean_ref[...] = mean
    logvar_ref[...] = logvar
    z_ref[...] = z
    y = jnp.dot(z, w_ref[...], preferred_element_type=jnp.float32)
    u3_ref[...] = dinv * y


def _tc4_body(deg_ref, p_ref, u3_ref, w_ref, b_ref, u4_ref):
    dinv = _dinv_of(deg_ref)
    hd = dinv * (p_ref[0] + p_ref[1] + u3_ref[...]) + b_ref[...]
    y = jnp.dot(hd, w_ref[...], preferred_element_type=jnp.float32)
    u4_ref[...] = dinv * y


def _tc5_body(deg_ref, p_ref, u4_ref, b_ref, o_ref):
    dinv = _dinv_of(deg_ref)
    o_ref[...] = dinv * (p_ref[0] + p_ref[1] + u4_ref[...]) + b_ref[...]


_GRID = (NPAD // BR,)


def _spec_deg():
    return pl.BlockSpec((NC, BR, DEGW), lambda i: (0, i, 0))


def _spec_p():
    return pl.BlockSpec((NC, BR, D), lambda i: (0, i, 0))


def _spec_rows(width=D):
    return pl.BlockSpec((BR, width), lambda i: (i, 0))


def _spec_w(k=D, n=D):
    return pl.BlockSpec((k, n), lambda i: (0, 0))


def _spec_b():
    return pl.BlockSpec((1, D), lambda i: (0, 0))


def _f32(*shape):
    return jax.ShapeDtypeStruct(shape, jnp.float32)


def kernel(feature, edge_index, W_enc, b_enc, W_mean, b_mean, W_logvar,
           b_logvar, W_dec1, b_dec1, W_dec2, b_dec2):
    src_p = jnp.concatenate(
        [edge_index[0], jnp.full((EPAD - EE,), NN, jnp.int32)])
    dst_p = jnp.concatenate(
        [edge_index[1], jnp.full((EPAD - EE,), NN, jnp.int32)])
    x_p = jnp.pad(feature, ((0, NPAD - NN), (0, 0)))
    noise = jax.random.normal(jax.random.key(1), (NN, LD), jnp.float32)
    noise_p = jnp.pad(noise, ((0, NPAD - NN), (0, 0)))
    w_cat = jnp.concatenate([W_mean, W_logvar], axis=1)
    b_cat = jnp.concatenate([b_mean, b_logvar]).reshape(1, D)

    deg3 = _deg_call(dst_p).reshape(NC, NPAD, DEGW)

    u1 = pl.pallas_call(
        _tc1_body, grid=_GRID,
        in_specs=[_spec_deg(), _spec_rows(), _spec_w()],
        out_specs=_spec_rows(), out_shape=_f32(NPAD, D),
    )(deg3, x_p, W_enc)

    p1 = _spmm_call(src_p, dst_p, u1)

    u2 = pl.pallas_call(
        _tc2_body, grid=_GRID,
        in_specs=[_spec_deg(), _spec_p(), _spec_rows(), _spec_w(), _spec_b()],
        out_specs=_spec_rows(), out_shape=_f32(NPAD, D),
    )(deg3, p1, u1, w_cat, b_enc.reshape(1, D))

    p2 = _spmm_call(src_p, dst_p, u2)

    mean, logvar, z, u3 = pl.pallas_call(
        _tc3_body, grid=_GRID,
        in_specs=[_spec_deg(), _spec_p(), _spec_rows(), _spec_b(),
                  _spec_rows(LD), _spec_w(LD, D)],
        out_specs=[_spec_rows(LD), _spec_rows(LD), _spec_rows(LD),
                   _spec_rows()],
        out_shape=[_f32(NPAD, LD), _f32(NPAD, LD), _f32(NPAD, LD),
                   _f32(NPAD, D)],
    )(deg3, p2, u2, b_cat, noise_p, W_dec1)

    p3 = _spmm_call(src_p, dst_p, u3)

    u4 = pl.pallas_call(
        _tc4_body, grid=_GRID,
        in_specs=[_spec_deg(), _spec_p(), _spec_rows(), _spec_w(), _spec_b()],
        out_specs=_spec_rows(), out_shape=_f32(NPAD, D),
    )(deg3, p3, u3, W_dec2, b_dec1.reshape(1, D))

    p4 = _spmm_call(src_p, dst_p, u4)

    out = pl.pallas_call(
        _tc5_body, grid=_GRID,
        in_specs=[_spec_deg(), _spec_p(), _spec_rows(), _spec_b()],
        out_specs=_spec_rows(), out_shape=_f32(NPAD, D),
    )(deg3, p4, u4, b_dec2.reshape(1, D))

    return (z[:NN], mean[:NN], logvar[:NN], out[:NN])


# speedup vs baseline: 3.3992x; 3.3992x over previous
import functools

import jax
import jax.numpy as jnp
from jax import lax
from jax.experimental import pallas as pl
from jax.experimental.pallas import tpu as pltpu
from jax.experimental.pallas import tpu_sc as plsc

NN = 10000
EE = 320000
D = 128
LD = 64

NC = 2
NS = 16
NW = NC * NS

EB = 128
NB = 80
EPT = NB * EB
EPAD = EPT * NW
EPC = EPAD // NC

NPAD = 10240
RPT = NPAD // NS
DEGW = 16

BR = 1024


def _sc_mesh():
    return plsc.VectorSubcoreMesh(
        core_axis_name="c", subcore_axis_name="s", num_cores=NC, num_subcores=NS)


def _deg_body(dst_h, out_h, acc, idx_d, ones_b, zero_b):
    c = lax.axis_index("c")
    s = lax.axis_index("s")
    one_v = jnp.ones((16,), jnp.float32)
    zero_v = jnp.zeros((16,), jnp.float32)
    for r in range(EB):
        ones_b[r, :] = one_v
        zero_b[r, :] = zero_v
    r0 = s * RPT
    def zloop(i, carry):
        pltpu.sync_copy(zero_b, acc.at[pl.ds(r0 + i * EB, EB)])
        return carry
    lax.fori_loop(0, RPT // EB, zloop, 0)
    plsc.subcore_barrier()
    base = c * EPC + s * EPT
    def eloop(b, carry):
        pltpu.sync_copy(dst_h.at[pl.ds(base + b * EB, EB)], idx_d)
        pltpu.sync_copy(ones_b, acc.at[idx_d], add=True)
        return carry
    lax.fori_loop(0, NB, eloop, 0)
    plsc.subcore_barrier()
    def oloop(k, carry):
        rr = r0 + k * EB
        pltpu.sync_copy(acc.at[pl.ds(rr, EB)], zero_b)
        pltpu.sync_copy(zero_b, out_h.at[pl.ds(c * NPAD + rr, EB)])
        return carry
    lax.fori_loop(0, RPT // EB, oloop, 0)


@jax.jit
def _deg_call(dst_p):
    return pl.kernel(
        _deg_body,
        out_type=jax.ShapeDtypeStruct((NC * NPAD, DEGW), jnp.float32),
        mesh=_sc_mesh(),
        scratch_types=[
            pltpu.VMEM_SHARED((NPAD, DEGW), jnp.float32),
            pltpu.VMEM((EB,), jnp.int32),
            pltpu.VMEM((EB, DEGW), jnp.float32),
            pltpu.VMEM((EB, DEGW), jnp.float32),
        ],
    )(dst_p)


EROWS = EPAD // EB


def _spmm_body(src_h, dst_h, u_h, out_h, acc, src_b, d_a, d_b, rows_a, rows_b,
               sem_a, sem_b, dsem_a, dsem_b):
    c = lax.axis_index("c")
    s = lax.axis_index("s")
    zero_v = jnp.zeros((16,), jnp.float32)
    for r in range(16):
        for j in range(8):
            rows_a[r, pl.ds(j * 16, 16)] = zero_v
    r0 = s * RPT
    zsrc = rows_a.at[pl.ds(0, 16)]
    def zloop(i, carry):
        pltpu.sync_copy(zsrc, acc.at[pl.ds(r0 + i * 16, 16)])
        return carry
    lax.fori_loop(0, RPT // 16, zloop, 0)
    base = c * EPC + s * EPT
    pltpu.sync_copy(src_h.at[pl.ds(base, EPT)], src_b)
    plsc.subcore_barrier()

    def gstart(b, rbuf, sem):
        pltpu.async_copy(u_h.at[src_b.at[pl.ds(b * EB, EB)]], rbuf, sem)

    def gwait(rbuf, sem):
        pltpu.make_async_copy(u_h.at[src_b.at[pl.ds(0, EB)]], rbuf, sem).wait()

    def dstart(b, dbuf, dsem):
        pltpu.async_copy(dst_h.at[pl.ds(base + b * EB, EB)], dbuf, dsem)

    def dwait(dbuf, dsem):
        pltpu.make_async_copy(dst_h.at[pl.ds(base, EB)], dbuf, dsem).wait()

    def scat(rbuf, dbuf):
        pltpu.sync_copy(rbuf, acc.at[dbuf], add=True)

    gstart(0, rows_a, sem_a)
    dstart(0, d_a, dsem_a)
    def eloop(g, carry):
        b0 = 2 * g
        gstart(b0 + 1, rows_b, sem_b)
        dstart(b0 + 1, d_b, dsem_b)
        gwait(rows_a, sem_a)
        dwait(d_a, dsem_a)
        scat(rows_a, d_a)
        gstart(b0 + 2, rows_a, sem_a)
        dstart(b0 + 2, d_a, dsem_a)
        gwait(rows_b, sem_b)
        dwait(d_b, dsem_b)
        scat(rows_b, d_b)
        return carry
    lax.fori_loop(0, NB // 2 - 1, eloop, 0)
    gstart(NB - 1, rows_b, sem_b)
    dstart(NB - 1, d_b, dsem_b)
    gwait(rows_a, sem_a)
    dwait(d_a, dsem_a)
    scat(rows_a, d_a)
    gwait(rows_b, sem_b)
    dwait(d_b, dsem_b)
    scat(rows_b, d_b)
    plsc.subcore_barrier()
    def oloop(k, carry):
        rr = r0 + k * EB
        pltpu.sync_copy(acc.at[pl.ds(rr, EB)], rows_a)
        pltpu.sync_copy(rows_a, out_h.at[pl.ds(c * NPAD + rr, EB)])
        return carry
    lax.fori_loop(0, RPT // EB, oloop, 0)


@jax.jit
def _spmm_call(src_p, dst_p, u):
    out = pl.kernel(
        _spmm_body,
        out_type=jax.ShapeDtypeStruct((NC * NPAD, D), jnp.float32),
        mesh=_sc_mesh(),
        scratch_types=[
            pltpu.VMEM_SHARED((NPAD, D), jnp.float32),
            pltpu.VMEM((NB * EB,), jnp.int32),
            pltpu.VMEM((EB,), jnp.int32),
            pltpu.VMEM((EB,), jnp.int32),
            pltpu.VMEM((EB, D), jnp.float32),
            pltpu.VMEM((EB, D), jnp.float32),
            pltpu.SemaphoreType.DMA,
            pltpu.SemaphoreType.DMA,
            pltpu.SemaphoreType.DMA,
            pltpu.SemaphoreType.DMA,
        ],
    )(src_p, dst_p, u)
    return out.reshape(NC, NPAD, D)


def _dinv_of(deg_ref):
    deg = deg_ref[0] + deg_ref[1]
    return lax.rsqrt(deg[:, :1] + 1.0)


def _tc1_body(deg_ref, x_ref, w_ref, u_ref):
    dinv = _dinv_of(deg_ref)
    y = jnp.dot(x_ref[...], w_ref[...], preferred_element_type=jnp.float32)
    u_ref[...] = dinv * y


def _tc2_body(deg_ref, p_ref, u1_ref, w_ref, b_ref, u2_ref):
    dinv = _dinv_of(deg_ref)
    agg = p_ref[0] + p_ref[1] + u1_ref[...]
    h = dinv * agg + b_ref[...]
    y = jnp.dot(h, w_ref[...], preferred_element_type=jnp.float32)
    u2_ref[...] = dinv * y


def _tc3_body(deg_ref, p_ref, u2_ref, b_ref, n_ref, w_ref,
              mean_ref, logvar_ref, z_ref, u3_ref):
    dinv = _dinv_of(deg_ref)
    ml = dinv * (p_ref[0] + p_ref[1] + u2_ref[...]) + b_ref[...]
    mean = ml[:, :LD]
    logvar = ml[:, LD:]
    z = n_ref[...] * jnp.exp(0.5 * logvar) + mean
    mean_ref[...] = mean
    logvar_ref[...] = logvar
    z_ref[...] = z
    y = jnp.dot(z, w_ref[...], preferred_element_type=jnp.float32)
    u3_ref[...] = dinv * y


def _tc4_body(deg_ref, p_ref, u3_ref, w_ref, b_ref, u4_ref):
    dinv = _dinv_of(deg_ref)
    hd = dinv * (p_ref[0] + p_ref[1] + u3_ref[...]) + b_ref[...]
    y = jnp.dot(hd, w_ref[...], preferred_element_type=jnp.float32)
    u4_ref[...] = dinv * y


def _tc5_body(deg_ref, p_ref, u4_ref, b_ref, o_ref):
    dinv = _dinv_of(deg_ref)
    o_ref[...] = dinv * (p_ref[0] + p_ref[1] + u4_ref[...]) + b_ref[...]


_GRID = (NPAD // BR,)


def _spec_deg():
    return pl.BlockSpec((NC, BR, DEGW), lambda i: (0, i, 0))


def _spec_p():
    return pl.BlockSpec((NC, BR, D), lambda i: (0, i, 0))


def _spec_rows(width=D):
    return pl.BlockSpec((BR, width), lambda i: (i, 0))


def _spec_w(k=D, n=D):
    return pl.BlockSpec((k, n), lambda i: (0, 0))


def _spec_b():
    return pl.BlockSpec((1, D), lambda i: (0, 0))


def _f32(*shape):
    return jax.ShapeDtypeStruct(shape, jnp.float32)


def kernel(feature, edge_index, W_enc, b_enc, W_mean, b_mean, W_logvar,
           b_logvar, W_dec1, b_dec1, W_dec2, b_dec2):
    pad_idx = NN + (jnp.arange(EPAD - EE, dtype=jnp.int32) % (NPAD - NN))
    src_p = jnp.concatenate([edge_index[0], pad_idx])
    dst_p = jnp.concatenate([edge_index[1], pad_idx])
    x_p = jnp.pad(feature, ((0, NPAD - NN), (0, 0)))
    noise = jax.random.normal(jax.random.key(1), (NN, LD), jnp.float32)
    noise_p = jnp.pad(noise, ((0, NPAD - NN), (0, 0)))
    w_cat = jnp.concatenate([W_mean, W_logvar], axis=1)
    b_cat = jnp.concatenate([b_mean, b_logvar]).reshape(1, D)

    deg3 = _deg_call(dst_p).reshape(NC, NPAD, DEGW)

    u1 = pl.pallas_call(
        _tc1_body, grid=_GRID,
        in_specs=[_spec_deg(), _spec_rows(), _spec_w()],
        out_specs=_spec_rows(), out_shape=_f32(NPAD, D),
    )(deg3, x_p, W_enc)

    p1 = _spmm_call(src_p, dst_p, u1)

    u2 = pl.pallas_call(
        _tc2_body, grid=_GRID,
        in_specs=[_spec_deg(), _spec_p(), _spec_rows(), _spec_w(), _spec_b()],
        out_specs=_spec_rows(), out_shape=_f32(NPAD, D),
    )(deg3, p1, u1, w_cat, b_enc.reshape(1, D))

    p2 = _spmm_call(src_p, dst_p, u2)

    mean, logvar, z, u3 = pl.pallas_call(
        _tc3_body, grid=_GRID,
        in_specs=[_spec_deg(), _spec_p(), _spec_rows(), _spec_b(),
                  _spec_rows(LD), _spec_w(LD, D)],
        out_specs=[_spec_rows(LD), _spec_rows(LD), _spec_rows(LD),
                   _spec_rows()],
        out_shape=[_f32(NPAD, LD), _f32(NPAD, LD), _f32(NPAD, LD),
                   _f32(NPAD, D)],
    )(deg3, p2, u2, b_cat, noise_p, W_dec1)

    p3 = _spmm_call(src_p, dst_p, u3)

    u4 = pl.pallas_call(
        _tc4_body, grid=_GRID,
        in_specs=[_spec_deg(), _spec_p(), _spec_rows(), _spec_w(), _spec_b()],
        out_specs=_spec_rows(), out_shape=_f32(NPAD, D),
    )(deg3, p3, u3, W_dec2, b_dec1.reshape(1, D))

    p4 = _spmm_call(src_p, dst_p, u4)

    out = pl.pallas_call(
        _tc5_body, grid=_GRID,
        in_specs=[_spec_deg(), _spec_p(), _spec_rows(), _spec_b()],
        out_specs=_spec_rows(), out_shape=_f32(NPAD, D),
    )(deg3, p4, u4, b_dec2.reshape(1, D))

    return (z[:NN], mean[:NN], logvar[:NN], out[:NN])


# speedup vs baseline: 3.7376x; 1.0996x over previous
import functools

import jax
import jax.numpy as jnp
from jax import lax
from jax.experimental import pallas as pl
from jax.experimental.pallas import tpu as pltpu
from jax.experimental.pallas import tpu_sc as plsc

NN = 10000
EE = 320000
D = 128
LD = 64

NC = 2
NS = 16
NW = NC * NS

EB = 128
NB = 80
EPT = NB * EB
EPAD = EPT * NW
EPC = EPAD // NC

NPAD = 10240
RPT = NPAD // NS
DEGW = 16

BR = 1024


def _sc_mesh():
    return plsc.VectorSubcoreMesh(
        core_axis_name="c", subcore_axis_name="s", num_cores=NC, num_subcores=NS)


def _deg_body(dst_h, out_h, acc, idx_d, ones_b, zero_b):
    c = lax.axis_index("c")
    s = lax.axis_index("s")
    one_v = jnp.ones((16,), jnp.float32)
    zero_v = jnp.zeros((16,), jnp.float32)
    for r in range(EB):
        ones_b[r, :] = one_v
        zero_b[r, :] = zero_v
    r0 = s * RPT
    def zloop(i, carry):
        pltpu.sync_copy(zero_b, acc.at[pl.ds(r0 + i * EB, EB)])
        return carry
    lax.fori_loop(0, RPT // EB, zloop, 0)
    plsc.subcore_barrier()
    base = c * EPC + s * EPT
    def eloop(b, carry):
        pltpu.sync_copy(dst_h.at[pl.ds(base + b * EB, EB)], idx_d)
        pltpu.sync_copy(ones_b, acc.at[idx_d], add=True)
        return carry
    lax.fori_loop(0, NB, eloop, 0)
    plsc.subcore_barrier()
    def oloop(k, carry):
        rr = r0 + k * EB
        pltpu.sync_copy(acc.at[pl.ds(rr, EB)], zero_b)
        pltpu.sync_copy(zero_b, out_h.at[pl.ds(c * NPAD + rr, EB)])
        return carry
    lax.fori_loop(0, RPT // EB, oloop, 0)


@jax.jit
def _deg_call(dst_p):
    return pl.kernel(
        _deg_body,
        out_type=jax.ShapeDtypeStruct((NC * NPAD, DEGW), jnp.float32),
        mesh=_sc_mesh(),
        scratch_types=[
            pltpu.VMEM_SHARED((NPAD, DEGW), jnp.float32),
            pltpu.VMEM((EB,), jnp.int32),
            pltpu.VMEM((EB, DEGW), jnp.float32),
            pltpu.VMEM((EB, DEGW), jnp.float32),
        ],
    )(dst_p)


EROWS = EPAD // EB


GB = 64
GNB = EPT // GB
NBUF = 4


def _spmm_body(src_h, dst_h, u_h, out_h, acc, src_b, dbufs, rbufs, gsems,
               dsems):
    c = lax.axis_index("c")
    s = lax.axis_index("s")
    zero_v = jnp.zeros((16,), jnp.float32)
    for r in range(16):
        for j in range(8):
            rbufs[0][r, pl.ds(j * 16, 16)] = zero_v
    r0 = s * RPT
    zsrc = rbufs[0].at[pl.ds(0, 16)]
    def zloop(i, carry):
        pltpu.sync_copy(zsrc, acc.at[pl.ds(r0 + i * 16, 16)])
        return carry
    lax.fori_loop(0, RPT // 16, zloop, 0)
    base = c * EPC + s * EPT
    pltpu.sync_copy(src_h.at[pl.ds(base, EPT)], src_b)
    plsc.subcore_barrier()

    def start(b, k):
        pltpu.async_copy(u_h.at[src_b.at[pl.ds(b * GB, GB)]], rbufs[k],
                         gsems[k])
        pltpu.async_copy(dst_h.at[pl.ds(base + b * GB, GB)], dbufs[k],
                         dsems[k])

    def drain_scat(k):
        pltpu.make_async_copy(u_h.at[src_b.at[pl.ds(0, GB)]], rbufs[k],
                              gsems[k]).wait()
        pltpu.make_async_copy(dst_h.at[pl.ds(base, GB)], dbufs[k],
                              dsems[k]).wait()
        pltpu.sync_copy(rbufs[k], acc.at[dbufs[k]], add=True)

    for k in range(NBUF - 1):
        start(k, k)
    def eloop(g, carry):
        b0 = NBUF * g
        for k in range(NBUF):
            start(b0 + k + NBUF - 1, (k + NBUF - 1) % NBUF)
            drain_scat(k)
        return carry
    lax.fori_loop(0, GNB // NBUF - 1, eloop, 0)
    start(GNB - 1, NBUF - 1)
    for k in range(NBUF):
        drain_scat(k)
    plsc.subcore_barrier()
    def oloop(k, carry):
        rr = r0 + k * GB
        pltpu.sync_copy(acc.at[pl.ds(rr, GB)], rbufs[0])
        pltpu.sync_copy(rbufs[0], out_h.at[pl.ds(c * NPAD + rr, GB)])
        return carry
    lax.fori_loop(0, RPT // GB, oloop, 0)


@jax.jit
def _spmm_call(src_p, dst_p, u):
    out = pl.kernel(
        _spmm_body,
        out_type=jax.ShapeDtypeStruct((NC * NPAD, D), jnp.float32),
        mesh=_sc_mesh(),
        scratch_types=[
            pltpu.VMEM_SHARED((NPAD, D), jnp.float32),
            pltpu.VMEM((NB * EB,), jnp.int32),
            [pltpu.VMEM((GB,), jnp.int32) for _ in range(NBUF)],
            [pltpu.VMEM((GB, D), jnp.float32) for _ in range(NBUF)],
            [pltpu.SemaphoreType.DMA for _ in range(NBUF)],
            [pltpu.SemaphoreType.DMA for _ in range(NBUF)],
        ],
    )(src_p, dst_p, u)
    return out.reshape(NC, NPAD, D)


def _dinv_of(deg_ref):
    deg = deg_ref[0] + deg_ref[1]
    return lax.rsqrt(deg[:, :1] + 1.0)


def _tc1_body(deg_ref, x_ref, w_ref, u_ref):
    dinv = _dinv_of(deg_ref)
    y = jnp.dot(x_ref[...], w_ref[...], preferred_element_type=jnp.float32)
    u_ref[...] = dinv * y


def _tc2_body(deg_ref, p_ref, u1_ref, w_ref, b_ref, u2_ref):
    dinv = _dinv_of(deg_ref)
    agg = p_ref[0] + p_ref[1] + u1_ref[...]
    h = dinv * agg + b_ref[...]
    y = jnp.dot(h, w_ref[...], preferred_element_type=jnp.float32)
    u2_ref[...] = dinv * y


def _tc3_body(deg_ref, p_ref, u2_ref, b_ref, n_ref, w_ref,
              mean_ref, logvar_ref, z_ref, u3_ref):
    dinv = _dinv_of(deg_ref)
    ml = dinv * (p_ref[0] + p_ref[1] + u2_ref[...]) + b_ref[...]
    mean = ml[:, :LD]
    logvar = ml[:, LD:]
    z = n_ref[...] * jnp.exp(0.5 * logvar) + mean
    mean_ref[...] = mean
    logvar_ref[...] = logvar
    z_ref[...] = z
    y = jnp.dot(z, w_ref[...], preferred_element_type=jnp.float32)
    u3_ref[...] = dinv * y


def _tc4_body(deg_ref, p_ref, u3_ref, w_ref, b_ref, u4_ref):
    dinv = _dinv_of(deg_ref)
    hd = dinv * (p_ref[0] + p_ref[1] + u3_ref[...]) + b_ref[...]
    y = jnp.dot(hd, w_ref[...], preferred_element_type=jnp.float32)
    u4_ref[...] = dinv * y


def _tc5_body(deg_ref, p_ref, u4_ref, b_ref, o_ref):
    dinv = _dinv_of(deg_ref)
    o_ref[...] = dinv * (p_ref[0] + p_ref[1] + u4_ref[...]) + b_ref[...]


_GRID = (NPAD // BR,)


def _spec_deg():
    return pl.BlockSpec((NC, BR, DEGW), lambda i: (0, i, 0))


def _spec_p():
    return pl.BlockSpec((NC, BR, D), lambda i: (0, i, 0))


def _spec_rows(width=D):
    return pl.BlockSpec((BR, width), lambda i: (i, 0))


def _spec_w(k=D, n=D):
    return pl.BlockSpec((k, n), lambda i: (0, 0))


def _spec_b():
    return pl.BlockSpec((1, D), lambda i: (0, 0))


def _f32(*shape):
    return jax.ShapeDtypeStruct(shape, jnp.float32)


def kernel(feature, edge_index, W_enc, b_enc, W_mean, b_mean, W_logvar,
           b_logvar, W_dec1, b_dec1, W_dec2, b_dec2):
    pad_idx = NN + (jnp.arange(EPAD - EE, dtype=jnp.int32) % (NPAD - NN))
    src_p = jnp.concatenate([edge_index[0], pad_idx])
    dst_p = jnp.concatenate([edge_index[1], pad_idx])
    x_p = jnp.pad(feature, ((0, NPAD - NN), (0, 0)))
    noise = jax.random.normal(jax.random.key(1), (NN, LD), jnp.float32)
    noise_p = jnp.pad(noise, ((0, NPAD - NN), (0, 0)))
    w_cat = jnp.concatenate([W_mean, W_logvar], axis=1)
    b_cat = jnp.concatenate([b_mean, b_logvar]).reshape(1, D)

    deg3 = _deg_call(dst_p).reshape(NC, NPAD, DEGW)

    u1 = pl.pallas_call(
        _tc1_body, grid=_GRID,
        in_specs=[_spec_deg(), _spec_rows(), _spec_w()],
        out_specs=_spec_rows(), out_shape=_f32(NPAD, D),
    )(deg3, x_p, W_enc)

    p1 = _spmm_call(src_p, dst_p, u1)

    u2 = pl.pallas_call(
        _tc2_body, grid=_GRID,
        in_specs=[_spec_deg(), _spec_p(), _spec_rows(), _spec_w(), _spec_b()],
        out_specs=_spec_rows(), out_shape=_f32(NPAD, D),
    )(deg3, p1, u1, w_cat, b_enc.reshape(1, D))

    p2 = _spmm_call(src_p, dst_p, u2)

    mean, logvar, z, u3 = pl.pallas_call(
        _tc3_body, grid=_GRID,
        in_specs=[_spec_deg(), _spec_p(), _spec_rows(), _spec_b(),
                  _spec_rows(LD), _spec_w(LD, D)],
        out_specs=[_spec_rows(LD), _spec_rows(LD), _spec_rows(LD),
                   _spec_rows()],
        out_shape=[_f32(NPAD, LD), _f32(NPAD, LD), _f32(NPAD, LD),
                   _f32(NPAD, D)],
    )(deg3, p2, u2, b_cat, noise_p, W_dec1)

    p3 = _spmm_call(src_p, dst_p, u3)

    u4 = pl.pallas_call(
        _tc4_body, grid=_GRID,
        in_specs=[_spec_deg(), _spec_p(), _spec_rows(), _spec_w(), _spec_b()],
        out_specs=_spec_rows(), out_shape=_f32(NPAD, D),
    )(deg3, p3, u3, W_dec2, b_dec1.reshape(1, D))

    p4 = _spmm_call(src_p, dst_p, u4)

    out = pl.pallas_call(
        _tc5_body, grid=_GRID,
        in_specs=[_spec_deg(), _spec_p(), _spec_rows(), _spec_b()],
        out_specs=_spec_rows(), out_shape=_f32(NPAD, D),
    )(deg3, p4, u4, b_dec2.reshape(1, D))

    return (z[:NN], mean[:NN], logvar[:NN], out[:NN])


# speedup vs baseline: 3.9012x; 1.0438x over previous
import functools

import jax
import jax.numpy as jnp
from jax import lax
from jax.experimental import pallas as pl
from jax.experimental.pallas import tpu as pltpu
from jax.experimental.pallas import tpu_sc as plsc

NN = 10000
EE = 320000
D = 128
LD = 64

NC = 2
NS = 16
NW = NC * NS

EB = 128
NB = 80
EPT = NB * EB
EPAD = EPT * NW
EPC = EPAD // NC

NPAD = 10240
RPT = NPAD // NS
DEGW = 16

BR = 1024


def _sc_mesh():
    return plsc.VectorSubcoreMesh(
        core_axis_name="c", subcore_axis_name="s", num_cores=NC, num_subcores=NS)


def _deg_body(dst_h, out_h, acc, idx_a, idx_b, ones_b, zero_b, sem_a, sem_b):
    c = lax.axis_index("c")
    s = lax.axis_index("s")
    one_v = jnp.ones((16,), jnp.float32)
    zero_v = jnp.zeros((16,), jnp.float32)
    for r in range(EB):
        ones_b[r, :] = one_v
        zero_b[r, :] = zero_v
    r0 = s * RPT
    def zloop(i, carry):
        pltpu.sync_copy(zero_b, acc.at[pl.ds(r0 + i * EB, EB)])
        return carry
    lax.fori_loop(0, RPT // EB, zloop, 0)
    plsc.subcore_barrier()
    base = c * EPC + s * EPT

    def istart(b, buf, sem):
        pltpu.async_copy(dst_h.at[pl.ds(base + b * EB, EB)], buf, sem)

    def iscat(buf, sem):
        pltpu.make_async_copy(dst_h.at[pl.ds(base, EB)], buf, sem).wait()
        pltpu.sync_copy(ones_b, acc.at[buf], add=True)

    istart(0, idx_a, sem_a)
    def eloop(g, carry):
        b0 = 2 * g
        istart(b0 + 1, idx_b, sem_b)
        iscat(idx_a, sem_a)
        istart(b0 + 2, idx_a, sem_a)
        iscat(idx_b, sem_b)
        return carry
    lax.fori_loop(0, NB // 2 - 1, eloop, 0)
    istart(NB - 1, idx_b, sem_b)
    iscat(idx_a, sem_a)
    iscat(idx_b, sem_b)
    plsc.subcore_barrier()
    def oloop(k, carry):
        rr = r0 + k * EB
        pltpu.sync_copy(acc.at[pl.ds(rr, EB)], zero_b)
        pltpu.sync_copy(zero_b, out_h.at[pl.ds(c * NPAD + rr, EB)])
        return carry
    lax.fori_loop(0, RPT // EB, oloop, 0)


@jax.jit
def _deg_call(dst_p):
    return pl.kernel(
        _deg_body,
        out_type=jax.ShapeDtypeStruct((NC * NPAD, DEGW), jnp.float32),
        mesh=_sc_mesh(),
        scratch_types=[
            pltpu.VMEM_SHARED((NPAD, DEGW), jnp.float32),
            pltpu.VMEM((EB,), jnp.int32),
            pltpu.VMEM((EB,), jnp.int32),
            pltpu.VMEM((EB, DEGW), jnp.float32),
            pltpu.VMEM((EB, DEGW), jnp.float32),
            pltpu.SemaphoreType.DMA,
            pltpu.SemaphoreType.DMA,
        ],
    )(dst_p)


EROWS = EPAD // EB


GB = 64
GNB = EPT // GB
NBUF = 4


def _spmm_body(src_h, dst_h, u_h, out_h, acc, src_b, dbufs, rbufs, gsems,
               dsems):
    c = lax.axis_index("c")
    s = lax.axis_index("s")
    zero_v = jnp.zeros((16,), jnp.float32)
    for r in range(16):
        for j in range(8):
            rbufs[0][r, pl.ds(j * 16, 16)] = zero_v
    r0 = s * RPT
    zsrc = rbufs[0].at[pl.ds(0, 16)]
    def zloop(i, carry):
        pltpu.sync_copy(zsrc, acc.at[pl.ds(r0 + i * 16, 16)])
        return carry
    lax.fori_loop(0, RPT // 16, zloop, 0)
    base = c * EPC + s * EPT
    pltpu.sync_copy(src_h.at[pl.ds(base, EPT)], src_b)
    plsc.subcore_barrier()

    def start(b, k):
        pltpu.async_copy(u_h.at[src_b.at[pl.ds(b * GB, GB)]], rbufs[k],
                         gsems[k])
        pltpu.async_copy(dst_h.at[pl.ds(base + b * GB, GB)], dbufs[k],
                         dsems[k])

    def drain_scat(k):
        pltpu.make_async_copy(u_h.at[src_b.at[pl.ds(0, GB)]], rbufs[k],
                              gsems[k]).wait()
        pltpu.make_async_copy(dst_h.at[pl.ds(base, GB)], dbufs[k],
                              dsems[k]).wait()
        pltpu.sync_copy(rbufs[k], acc.at[dbufs[k]], add=True)

    for k in range(NBUF - 1):
        start(k, k)
    def eloop(g, carry):
        b0 = NBUF * g
        for k in range(NBUF):
            start(b0 + k + NBUF - 1, (k + NBUF - 1) % NBUF)
            drain_scat(k)
        return carry
    lax.fori_loop(0, GNB // NBUF - 1, eloop, 0)
    start(GNB - 1, NBUF - 1)
    for k in range(NBUF):
        drain_scat(k)
    plsc.subcore_barrier()
    def oloop(k, carry):
        rr = r0 + k * GB
        pltpu.sync_copy(acc.at[pl.ds(rr, GB)], rbufs[0])
        pltpu.sync_copy(rbufs[0], out_h.at[pl.ds(c * NPAD + rr, GB)])
        return carry
    lax.fori_loop(0, RPT // GB, oloop, 0)


@jax.jit
def _spmm_call(src_p, dst_p, u):
    out = pl.kernel(
        _spmm_body,
        out_type=jax.ShapeDtypeStruct((NC * NPAD, D), jnp.float32),
        mesh=_sc_mesh(),
        scratch_types=[
            pltpu.VMEM_SHARED((NPAD, D), jnp.float32),
            pltpu.VMEM((NB * EB,), jnp.int32),
            [pltpu.VMEM((GB,), jnp.int32) for _ in range(NBUF)],
            [pltpu.VMEM((GB, D), jnp.float32) for _ in range(NBUF)],
            [pltpu.SemaphoreType.DMA for _ in range(NBUF)],
            [pltpu.SemaphoreType.DMA for _ in range(NBUF)],
        ],
    )(src_p, dst_p, u)
    return out.reshape(NC, NPAD, D)


def _dinv_of(deg_ref):
    deg = deg_ref[0] + deg_ref[1]
    return lax.rsqrt(deg[:, :1] + 1.0)


def _tc1_body(deg_ref, x_ref, w_ref, u_ref):
    dinv = _dinv_of(deg_ref)
    y = jnp.dot(x_ref[...], w_ref[...], preferred_element_type=jnp.float32)
    u_ref[...] = dinv * y


def _tc2_body(deg_ref, p_ref, u1_ref, w_ref, b_ref, u2_ref):
    dinv = _dinv_of(deg_ref)
    agg = p_ref[0] + p_ref[1] + u1_ref[...]
    h = dinv * agg + b_ref[...]
    y = jnp.dot(h, w_ref[...], preferred_element_type=jnp.float32)
    u2_ref[...] = dinv * y


def _tc3_body(deg_ref, p_ref, u2_ref, b_ref, n_ref, w_ref,
              mean_ref, logvar_ref, z_ref, u3_ref):
    dinv = _dinv_of(deg_ref)
    ml = dinv * (p_ref[0] + p_ref[1] + u2_ref[...]) + b_ref[...]
    mean = ml[:, :LD]
    logvar = ml[:, LD:]
    z = n_ref[...] * jnp.exp(0.5 * logvar) + mean
    mean_ref[...] = mean
    logvar_ref[...] = logvar
    z_ref[...] = z
    y = jnp.dot(z, w_ref[...], preferred_element_type=jnp.float32)
    u3_ref[...] = dinv * y


def _tc4_body(deg_ref, p_ref, u3_ref, w_ref, b_ref, u4_ref):
    dinv = _dinv_of(deg_ref)
    hd = dinv * (p_ref[0] + p_ref[1] + u3_ref[...]) + b_ref[...]
    y = jnp.dot(hd, w_ref[...], preferred_element_type=jnp.float32)
    u4_ref[...] = dinv * y


def _tc5_body(deg_ref, p_ref, u4_ref, b_ref, o_ref):
    dinv = _dinv_of(deg_ref)
    o_ref[...] = dinv * (p_ref[0] + p_ref[1] + u4_ref[...]) + b_ref[...]


_GRID = (NPAD // BR,)


def _spec_deg():
    return pl.BlockSpec((NC, BR, DEGW), lambda i: (0, i, 0))


def _spec_p():
    return pl.BlockSpec((NC, BR, D), lambda i: (0, i, 0))


def _spec_rows(width=D):
    return pl.BlockSpec((BR, width), lambda i: (i, 0))


def _spec_w(k=D, n=D):
    return pl.BlockSpec((k, n), lambda i: (0, 0))


def _spec_b():
    return pl.BlockSpec((1, D), lambda i: (0, 0))


def _f32(*shape):
    return jax.ShapeDtypeStruct(shape, jnp.float32)


def kernel(feature, edge_index, W_enc, b_enc, W_mean, b_mean, W_logvar,
           b_logvar, W_dec1, b_dec1, W_dec2, b_dec2):
    pad_idx = NN + (jnp.arange(EPAD - EE, dtype=jnp.int32) % (NPAD - NN))
    src_p = jnp.concatenate([edge_index[0], pad_idx])
    dst_p = jnp.concatenate([edge_index[1], pad_idx])
    x_p = jnp.pad(feature, ((0, NPAD - NN), (0, 0)))
    noise = jax.random.normal(jax.random.key(1), (NN, LD), jnp.float32)
    noise_p = jnp.pad(noise, ((0, NPAD - NN), (0, 0)))
    w_cat = jnp.concatenate([W_mean, W_logvar], axis=1)
    b_cat = jnp.concatenate([b_mean, b_logvar]).reshape(1, D)

    deg3 = _deg_call(dst_p).reshape(NC, NPAD, DEGW)

    u1 = pl.pallas_call(
        _tc1_body, grid=_GRID,
        in_specs=[_spec_deg(), _spec_rows(), _spec_w()],
        out_specs=_spec_rows(), out_shape=_f32(NPAD, D),
    )(deg3, x_p, W_enc)

    p1 = _spmm_call(src_p, dst_p, u1)

    u2 = pl.pallas_call(
        _tc2_body, grid=_GRID,
        in_specs=[_spec_deg(), _spec_p(), _spec_rows(), _spec_w(), _spec_b()],
        out_specs=_spec_rows(), out_shape=_f32(NPAD, D),
    )(deg3, p1, u1, w_cat, b_enc.reshape(1, D))

    p2 = _spmm_call(src_p, dst_p, u2)

    mean, logvar, z, u3 = pl.pallas_call(
        _tc3_body, grid=_GRID,
        in_specs=[_spec_deg(), _spec_p(), _spec_rows(), _spec_b(),
                  _spec_rows(LD), _spec_w(LD, D)],
        out_specs=[_spec_rows(LD), _spec_rows(LD), _spec_rows(LD),
                   _spec_rows()],
        out_shape=[_f32(NPAD, LD), _f32(NPAD, LD), _f32(NPAD, LD),
                   _f32(NPAD, D)],
    )(deg3, p2, u2, b_cat, noise_p, W_dec1)

    p3 = _spmm_call(src_p, dst_p, u3)

    u4 = pl.pallas_call(
        _tc4_body, grid=_GRID,
        in_specs=[_spec_deg(), _spec_p(), _spec_rows(), _spec_w(), _spec_b()],
        out_specs=_spec_rows(), out_shape=_f32(NPAD, D),
    )(deg3, p3, u3, W_dec2, b_dec1.reshape(1, D))

    p4 = _spmm_call(src_p, dst_p, u4)

    out = pl.pallas_call(
        _tc5_body, grid=_GRID,
        in_specs=[_spec_deg(), _spec_p(), _spec_rows(), _spec_b()],
        out_specs=_spec_rows(), out_shape=_f32(NPAD, D),
    )(deg3, p4, u4, b_dec2.reshape(1, D))

    return (z[:NN], mean[:NN], logvar[:NN], out[:NN])


# speedup vs baseline: 3.9357x; 1.0088x over previous
import functools

import jax
import jax.numpy as jnp
from jax import lax
from jax.experimental import pallas as pl
from jax.experimental.pallas import tpu as pltpu
from jax.experimental.pallas import tpu_sc as plsc

NN = 10000
EE = 320000
D = 128
LD = 64

NC = 2
NS = 16
NW = NC * NS

EB = 128
NB = 80
EPT = NB * EB
EPAD = EPT * NW
EPC = EPAD // NC

NPAD = 10240
RPT = NPAD // NS
DEGW = 16

BR = 1024


def _sc_mesh():
    return plsc.VectorSubcoreMesh(
        core_axis_name="c", subcore_axis_name="s", num_cores=NC, num_subcores=NS)


def _deg_body(dst_h, out_h, acc, idx_a, idx_b, ones_b, zero_b, sem_a, sem_b):
    c = lax.axis_index("c")
    s = lax.axis_index("s")
    one_v = jnp.ones((16,), jnp.float32)
    zero_v = jnp.zeros((16,), jnp.float32)
    for r in range(EB):
        ones_b[r, :] = one_v
        zero_b[r, :] = zero_v
    r0 = s * RPT
    def zloop(i, carry):
        pltpu.sync_copy(zero_b, acc.at[pl.ds(r0 + i * EB, EB)])
        return carry
    lax.fori_loop(0, RPT // EB, zloop, 0)
    plsc.subcore_barrier()
    base = c * EPC + s * EPT

    def istart(b, buf, sem):
        pltpu.async_copy(dst_h.at[pl.ds(base + b * EB, EB)], buf, sem)

    def iscat(buf, sem):
        pltpu.make_async_copy(dst_h.at[pl.ds(base, EB)], buf, sem).wait()
        pltpu.sync_copy(ones_b, acc.at[buf], add=True)

    istart(0, idx_a, sem_a)
    def eloop(g, carry):
        b0 = 2 * g
        istart(b0 + 1, idx_b, sem_b)
        iscat(idx_a, sem_a)
        istart(b0 + 2, idx_a, sem_a)
        iscat(idx_b, sem_b)
        return carry
    lax.fori_loop(0, NB // 2 - 1, eloop, 0)
    istart(NB - 1, idx_b, sem_b)
    iscat(idx_a, sem_a)
    iscat(idx_b, sem_b)
    plsc.subcore_barrier()
    def oloop(k, carry):
        rr = r0 + k * EB
        pltpu.sync_copy(acc.at[pl.ds(rr, EB)], zero_b)
        pltpu.sync_copy(zero_b, out_h.at[pl.ds(c * NPAD + rr, EB)])
        return carry
    lax.fori_loop(0, RPT // EB, oloop, 0)


@jax.jit
def _deg_call(dst_p):
    return pl.kernel(
        _deg_body,
        out_type=jax.ShapeDtypeStruct((NC * NPAD, DEGW), jnp.float32),
        mesh=_sc_mesh(),
        scratch_types=[
            pltpu.VMEM_SHARED((NPAD, DEGW), jnp.float32),
            pltpu.VMEM((EB,), jnp.int32),
            pltpu.VMEM((EB,), jnp.int32),
            pltpu.VMEM((EB, DEGW), jnp.float32),
            pltpu.VMEM((EB, DEGW), jnp.float32),
            pltpu.SemaphoreType.DMA,
            pltpu.SemaphoreType.DMA,
        ],
    )(dst_p)


EROWS = EPAD // EB


GB = 64
GNB = EPT // GB
NBUF = 4


def _spmm_body(src_h, dst_h, u_h, out_h, acc, src_b, dbufs, rbufs, gsems,
               dsems):
    c = lax.axis_index("c")
    s = lax.axis_index("s")
    zero_v = jnp.zeros((16,), jnp.float32)
    for r in range(16):
        for j in range(8):
            rbufs[0][r, pl.ds(j * 16, 16)] = zero_v
    r0 = s * RPT
    zsrc = rbufs[0].at[pl.ds(0, 16)]
    def zloop(i, carry):
        pltpu.sync_copy(zsrc, acc.at[pl.ds(r0 + i * 16, 16)])
        return carry
    lax.fori_loop(0, RPT // 16, zloop, 0)
    base = c * EPC + s * EPT
    pltpu.sync_copy(src_h.at[pl.ds(base, EPT)], src_b)
    plsc.subcore_barrier()

    def start(b, k):
        pltpu.async_copy(u_h.at[src_b.at[pl.ds(b * GB, GB)]], rbufs[k],
                         gsems[k])
        pltpu.async_copy(dst_h.at[pl.ds(base + b * GB, GB)], dbufs[k],
                         dsems[k])

    def drain_scat(k):
        pltpu.make_async_copy(u_h.at[src_b.at[pl.ds(0, GB)]], rbufs[k],
                              gsems[k]).wait()
        pltpu.make_async_copy(dst_h.at[pl.ds(base, GB)], dbufs[k],
                              dsems[k]).wait()
        pltpu.sync_copy(rbufs[k], acc.at[dbufs[k]], add=True)

    for k in range(NBUF - 1):
        start(k, k)
    def eloop(g, carry):
        b0 = NBUF * g
        for k in range(NBUF):
            start(b0 + k + NBUF - 1, (k + NBUF - 1) % NBUF)
            drain_scat(k)
        return carry
    lax.fori_loop(0, GNB // NBUF - 1, eloop, 0)
    start(GNB - 1, NBUF - 1)
    for k in range(NBUF):
        drain_scat(k)
    plsc.subcore_barrier()
    pltpu.sync_copy(acc.at[pl.ds(r0, RPT)],
                    out_h.at[pl.ds(c * NPAD + r0, RPT)])


@jax.jit
def _spmm_call(src_p, dst_p, u):
    out = pl.kernel(
        _spmm_body,
        out_type=jax.ShapeDtypeStruct((NC * NPAD, D), jnp.float32),
        mesh=_sc_mesh(),
        scratch_types=[
            pltpu.VMEM_SHARED((NPAD, D), jnp.float32),
            pltpu.VMEM((NB * EB,), jnp.int32),
            [pltpu.VMEM((GB,), jnp.int32) for _ in range(NBUF)],
            [pltpu.VMEM((GB, D), jnp.float32) for _ in range(NBUF)],
            [pltpu.SemaphoreType.DMA for _ in range(NBUF)],
            [pltpu.SemaphoreType.DMA for _ in range(NBUF)],
        ],
    )(src_p, dst_p, u)
    return out.reshape(NC, NPAD, D)


def _dinv_of(deg_ref):
    deg = deg_ref[0] + deg_ref[1]
    return lax.rsqrt(deg[:, :1] + 1.0)


def _tc1_body(deg_ref, x_ref, w_ref, u_ref):
    dinv = _dinv_of(deg_ref)
    y = jnp.dot(x_ref[...], w_ref[...], preferred_element_type=jnp.float32)
    u_ref[...] = dinv * y


def _tc2_body(deg_ref, p_ref, u1_ref, w_ref, b_ref, u2_ref):
    dinv = _dinv_of(deg_ref)
    agg = p_ref[0] + p_ref[1] + u1_ref[...]
    h = dinv * agg + b_ref[...]
    y = jnp.dot(h, w_ref[...], preferred_element_type=jnp.float32)
    u2_ref[...] = dinv * y


def _tc3_body(deg_ref, p_ref, u2_ref, b_ref, n_ref, w_ref,
              mean_ref, logvar_ref, z_ref, u3_ref):
    dinv = _dinv_of(deg_ref)
    ml = dinv * (p_ref[0] + p_ref[1] + u2_ref[...]) + b_ref[...]
    mean = ml[:, :LD]
    logvar = ml[:, LD:]
    z = n_ref[...] * jnp.exp(0.5 * logvar) + mean
    mean_ref[...] = mean
    logvar_ref[...] = logvar
    z_ref[...] = z
    y = jnp.dot(z, w_ref[...], preferred_element_type=jnp.float32)
    u3_ref[...] = dinv * y


def _tc4_body(deg_ref, p_ref, u3_ref, w_ref, b_ref, u4_ref):
    dinv = _dinv_of(deg_ref)
    hd = dinv * (p_ref[0] + p_ref[1] + u3_ref[...]) + b_ref[...]
    y = jnp.dot(hd, w_ref[...], preferred_element_type=jnp.float32)
    u4_ref[...] = dinv * y


def _tc5_body(deg_ref, p_ref, u4_ref, b_ref, o_ref):
    dinv = _dinv_of(deg_ref)
    o_ref[...] = dinv * (p_ref[0] + p_ref[1] + u4_ref[...]) + b_ref[...]


_GRID = (NPAD // BR,)


def _spec_deg():
    return pl.BlockSpec((NC, BR, DEGW), lambda i: (0, i, 0))


def _spec_p():
    return pl.BlockSpec((NC, BR, D), lambda i: (0, i, 0))


def _spec_rows(width=D):
    return pl.BlockSpec((BR, width), lambda i: (i, 0))


def _spec_w(k=D, n=D):
    return pl.BlockSpec((k, n), lambda i: (0, 0))


def _spec_b():
    return pl.BlockSpec((1, D), lambda i: (0, 0))


def _f32(*shape):
    return jax.ShapeDtypeStruct(shape, jnp.float32)


def kernel(feature, edge_index, W_enc, b_enc, W_mean, b_mean, W_logvar,
           b_logvar, W_dec1, b_dec1, W_dec2, b_dec2):
    pad_idx = NN + (jnp.arange(EPAD - EE, dtype=jnp.int32) % (NPAD - NN))
    src_p = jnp.concatenate([edge_index[0], pad_idx])
    dst_p = jnp.concatenate([edge_index[1], pad_idx])
    x_p = jnp.pad(feature, ((0, NPAD - NN), (0, 0)))
    noise = jax.random.normal(jax.random.key(1), (NN, LD), jnp.float32)
    noise_p = jnp.pad(noise, ((0, NPAD - NN), (0, 0)))
    w_cat = jnp.concatenate([W_mean, W_logvar], axis=1)
    b_cat = jnp.concatenate([b_mean, b_logvar]).reshape(1, D)

    deg3 = _deg_call(dst_p).reshape(NC, NPAD, DEGW)

    u1 = pl.pallas_call(
        _tc1_body, grid=_GRID,
        in_specs=[_spec_deg(), _spec_rows(), _spec_w()],
        out_specs=_spec_rows(), out_shape=_f32(NPAD, D),
    )(deg3, x_p, W_enc)

    p1 = _spmm_call(src_p, dst_p, u1)

    u2 = pl.pallas_call(
        _tc2_body, grid=_GRID,
        in_specs=[_spec_deg(), _spec_p(), _spec_rows(), _spec_w(), _spec_b()],
        out_specs=_spec_rows(), out_shape=_f32(NPAD, D),
    )(deg3, p1, u1, w_cat, b_enc.reshape(1, D))

    p2 = _spmm_call(src_p, dst_p, u2)

    mean, logvar, z, u3 = pl.pallas_call(
        _tc3_body, grid=_GRID,
        in_specs=[_spec_deg(), _spec_p(), _spec_rows(), _spec_b(),
                  _spec_rows(LD), _spec_w(LD, D)],
        out_specs=[_spec_rows(LD), _spec_rows(LD), _spec_rows(LD),
                   _spec_rows()],
        out_shape=[_f32(NPAD, LD), _f32(NPAD, LD), _f32(NPAD, LD),
                   _f32(NPAD, D)],
    )(deg3, p2, u2, b_cat, noise_p, W_dec1)

    p3 = _spmm_call(src_p, dst_p, u3)

    u4 = pl.pallas_call(
        _tc4_body, grid=_GRID,
        in_specs=[_spec_deg(), _spec_p(), _spec_rows(), _spec_w(), _spec_b()],
        out_specs=_spec_rows(), out_shape=_f32(NPAD, D),
    )(deg3, p3, u3, W_dec2, b_dec1.reshape(1, D))

    p4 = _spmm_call(src_p, dst_p, u4)

    out = pl.pallas_call(
        _tc5_body, grid=_GRID,
        in_specs=[_spec_deg(), _spec_p(), _spec_rows(), _spec_b()],
        out_specs=_spec_rows(), out_shape=_f32(NPAD, D),
    )(deg3, p4, u4, b_dec2.reshape(1, D))

    return (z[:NN], mean[:NN], logvar[:NN], out[:NN])
